# baseline (speedup 1.0000x reference)
import functools

import jax
import jax.numpy as jnp
import numpy as np
from jax import lax
from jax.experimental import pallas as pl
from jax.experimental.pallas import tpu as pltpu
from jax.experimental.pallas import tpu_sc as plsc

ALPHA = 0.2
LAMBDA = 1.0
BN_INV = float(1.0 / np.sqrt(1.0 + 1e-5))

NSUB = 16
LANE = 128
KB = 8

_SC_PARAMS = pltpu.CompilerParams(use_tc_tiling_on_sc=False)


def _fill_1d(ref, n, val):
    def body(k, _):
        ref[pl.ds(k * 16, 16)] = jnp.full((16,), val, jnp.float32)
        return _
    lax.fori_loop(0, n // 16, body, None)


def _fill_2d(ref, rows, cols, val):
    def body(r, _):
        for j in range(cols // 16):
            ref[r, pl.ds(j * 16, 16)] = jnp.full((16,), val, jnp.float32)
        return _
    lax.fori_loop(0, rows, body, None)


def _make_sc_degrees(NP, ROWS_T):
    STRIPE = NP // NSUB
    n_outer = ROWS_T // KB
    mesh = plsc.VectorSubcoreMesh(core_axis_name="c", subcore_axis_name="s")
    cnt_t = jax.ShapeDtypeStruct((NP,), jnp.float32)

    @functools.partial(
        pl.kernel,
        out_type=[cnt_t, cnt_t, cnt_t, cnt_t],
        mesh=mesh,
        compiler_params=_SC_PARAMS,
        scratch_types=[
            pltpu.VMEM((KB, LANE), jnp.int32),
            pltpu.VMEM((LANE,), jnp.float32),
            pltpu.VMEM((STRIPE,), jnp.float32),
            pltpu.VMEM_SHARED((NP,), jnp.float32),
            pltpu.VMEM_SHARED((NP,), jnp.float32),
        ],
    )
    def k(s1_h, d1_h, s2_h, d2_h, co1_h, ci1_h, co2_h, ci2_h,
          idx_v, ones_v, zb_v, acc_o, acc_i):
        c = lax.axis_index("c")
        t = lax.axis_index("s")
        _fill_1d(ones_v, LANE, 1.0)
        _fill_1d(zb_v, STRIPE, 0.0)
        pltpu.sync_copy(zb_v, acc_o.at[pl.ds(STRIPE * t, STRIPE)])
        pltpu.sync_copy(zb_v, acc_i.at[pl.ds(STRIPE * t, STRIPE)])
        plsc.subcore_barrier()

        def run(s_h, d_h, co_h, ci_h):
            def outer(i, carry):
                rb = t * ROWS_T + i * KB
                pltpu.sync_copy(s_h.at[pl.ds(rb, KB)], idx_v)
                for j in range(KB):
                    pltpu.sync_copy(ones_v, acc_o.at[idx_v.at[j]], add=True)
                pltpu.sync_copy(d_h.at[pl.ds(rb, KB)], idx_v)
                for j in range(KB):
                    pltpu.sync_copy(ones_v, acc_i.at[idx_v.at[j]], add=True)
                return carry
            lax.fori_loop(0, n_outer, outer, 0)
            plsc.subcore_barrier()
            pltpu.sync_copy(acc_o.at[pl.ds(STRIPE * t, STRIPE)],
                            co_h.at[pl.ds(STRIPE * t, STRIPE)])
            pltpu.sync_copy(acc_i.at[pl.ds(STRIPE * t, STRIPE)],
                            ci_h.at[pl.ds(STRIPE * t, STRIPE)])

        @pl.when(c == 0)
        def _():
            run(s1_h, d1_h, co1_h, ci1_h)

        @pl.when(c == 1)
        def _():
            run(s2_h, d2_h, co2_h, ci2_h)

    return k


def _make_sc_spmv(NP, D, ROWS_T):
    DH = D // 2
    STRIPE = NP // NSUB
    PH = 4
    PROWS = ROWS_T // PH
    NGRP = PROWS // 4
    mesh = plsc.VectorSubcoreMesh(core_axis_name="c", subcore_axis_name="s")

    @functools.partial(
        pl.kernel,
        out_type=jax.ShapeDtypeStruct((2, 2, NP, DH), jnp.float32),
        mesh=mesh,
        compiler_params=_SC_PARAMS,
        scratch_types=(
            [pltpu.VMEM((PROWS, LANE), jnp.int32),
             pltpu.VMEM((PROWS, LANE), jnp.int32),
             pltpu.VMEM((4 * LANE, DH), jnp.float32),
             pltpu.VMEM_SHARED((NP, DH), jnp.float32),
             pltpu.VMEM_SHARED((NP, DH), jnp.float32)]
            + [pltpu.SemaphoreType.DMA] * 8
        ),
    )
    def k(xs1a_h, xs1b_h, xs2a_h, xs2b_h, s1_h, d1_h, s2_h, d2_h, out_h,
          sidx_v, didx_v, rows_v, sxs, sacc,
          g0, g1, g2, g3, ss0, ss1, ss2, ss3):
        c = lax.axis_index("c")
        t = lax.axis_index("s")
        gsem = [g0, g1, g2, g3]
        ssem = [ss0, ss1, ss2, ss3]

        def slot(k_):
            return rows_v.at[pl.ds(k_ * LANE, LANE)]

        def gi(r, k_):
            pltpu.async_copy(sxs.at[sidx_v.at[r]], slot(k_), gsem[k_])

        def gw(r, k_):
            pltpu.make_async_copy(sxs.at[sidx_v.at[r]], slot(k_),
                                  gsem[k_]).wait()

        def si(r, k_):
            pltpu.async_copy(slot(k_), sacc.at[didx_v.at[r]], ssem[k_],
                             add=True)

        def sw(r, k_):
            pltpu.make_async_copy(slot(k_), sacc.at[didx_v.at[r]],
                                  ssem[k_]).wait()

        def run(xsa_h, xsb_h, s_h, d_h, mi):
            for half, xs_h in ((0, xsa_h), (1, xsb_h)):
                def gih(r, k_):
                    pltpu.async_copy(xs_h.at[sidx_v.at[r]], slot(k_),
                                     gsem[k_])

                def gwh(r, k_):
                    pltpu.make_async_copy(xs_h.at[sidx_v.at[r]], slot(k_),
                                          gsem[k_]).wait()
                pltpu.sync_copy(xs_h.at[pl.ds(STRIPE * t, STRIPE)],
                                sxs.at[pl.ds(STRIPE * t, STRIPE)])
                _fill_2d(rows_v, LANE, DH, 0.0)
                for q in range(STRIPE // LANE):
                    pltpu.sync_copy(
                        rows_v.at[pl.ds(0, LANE)],
                        sacc.at[pl.ds(STRIPE * t + LANE * q, LANE)])
                plsc.subcore_barrier()

                def phase(ph, carry):
                    rb = t * ROWS_T + ph * PROWS
                    pltpu.sync_copy(s_h.at[pl.ds(rb, PROWS)], sidx_v)
                    pltpu.sync_copy(d_h.at[pl.ds(rb, PROWS)], didx_v)
                    gi(0, 0)
                    gi(1, 1)
                    gi(2, 2)
                    gw(0, 0)
                    si(0, 0)
                    gih(3, 3)
                    gw(1, 1)
                    si(1, 1)

                    def group(g, cr):
                        for k_ in range(4):
                            r = g * 4 + k_
                            sw(r - 4, k_)
                            if k_ == 3:
                                gih(r, k_)
                            else:
                                gi(r, k_)
                            k2 = (k_ + 2) % 4
                            if k2 == 3:
                                gwh(r - 2, k2)
                            else:
                                gw(r - 2, k2)
                            si(r - 2, k2)
                        return cr
                    lax.fori_loop(1, NGRP, group, 0)
                    gw(PROWS - 2, 2)
                    si(PROWS - 2, 2)
                    gwh(PROWS - 1, 3)
                    si(PROWS - 1, 3)
                    for k_ in range(4):
                        sw(0, k_)
                    return carry
                lax.fori_loop(0, PH, phase, 0)
                plsc.subcore_barrier()
                pltpu.sync_copy(sacc.at[pl.ds(STRIPE * t, STRIPE)],
                                out_h.at[mi, half, pl.ds(STRIPE * t, STRIPE)])

        @pl.when(c == 0)
        def _():
            run(xs1a_h, xs1b_h, s1_h, d1_h, 0)

        @pl.when(c == 1)
        def _():
            run(xs2a_h, xs2b_h, s2_h, d2_h, 1)

    return k


def _row_block_spec(RB, D):
    return pl.BlockSpec((RB, D), lambda i: (i, 0))


def _full_spec(shape):
    return pl.BlockSpec(shape, lambda i: tuple(0 for _ in shape))


def _tc_prep(xp, co1, ci1, co2, ci2, NP, RB):
    D = xp.shape[1]
    grid = (NP // RB,)

    DH = D // 2

    def body(x_ref, co1_ref, ci1_ref, co2_ref, ci2_ref,
             xs1a_ref, xs1b_ref, xs2a_ref, xs2b_ref,
             dir1_ref, dir2_ref, dor1_ref, dor2_ref):
        x = x_ref[...]
        dor1 = lax.rsqrt(jnp.maximum(co1_ref[...], 1.0))
        dor2 = lax.rsqrt(jnp.maximum(co2_ref[...], 1.0))
        dir1_ref[...] = lax.rsqrt(jnp.maximum(ci1_ref[...], 1.0))
        dir2_ref[...] = lax.rsqrt(jnp.maximum(ci2_ref[...], 1.0))
        dor1_ref[...] = dor1
        dor2_ref[...] = dor2
        xs1 = x * dor1
        xs2 = x * dor2
        xs1a_ref[...] = xs1[:, :DH]
        xs1b_ref[...] = xs1[:, DH:]
        xs2a_ref[...] = xs2[:, :DH]
        xs2b_ref[...] = xs2[:, DH:]

    col = pl.BlockSpec((RB, 1), lambda i: (i, 0))
    halfspec = _row_block_spec(RB, DH)
    halfshape = jax.ShapeDtypeStruct((NP, DH), jnp.float32)
    return pl.pallas_call(
        body,
        grid=grid,
        in_specs=[_row_block_spec(RB, D), col, col, col, col],
        out_specs=[halfspec, halfspec, halfspec, halfspec,
                   col, col, col, col],
        out_shape=[
            halfshape, halfshape, halfshape, halfshape,
            jax.ShapeDtypeStruct((NP, 1), jnp.float32),
            jax.ShapeDtypeStruct((NP, 1), jnp.float32),
            jax.ShapeDtypeStruct((NP, 1), jnp.float32),
            jax.ShapeDtypeStruct((NP, 1), jnp.float32),
        ],
    )(xp, co1, ci1, co2, ci2)


def _tc_update(part, dir1, dir2, dor1, dor2, xp,
               W1, W2, g1, b1, g2, b2, beta, NP, RB):
    D = xp.shape[1]
    DH = D // 2
    grid = (NP // RB,)

    def body(p1a_ref, p1b_ref, p2a_ref, p2b_ref,
             dir1_ref, dir2_ref, dor1_ref, dor2_ref, x_ref,
             W1_ref, W2_ref, g1_ref, b1_ref, g2_ref, b2_ref,
             xs1a_ref, xs1b_ref, xs2a_ref, xs2b_ref):
        x = x_ref[...]

        def one(pa_ref, pb_ref, dirr, dorr, W_ref, g_ref, b_ref,
                oa_ref, ob_ref):
            p = jnp.concatenate([pa_ref[...].reshape(RB, DH),
                                 pb_ref[...].reshape(RB, DH)], axis=1)
            h = (1.0 - ALPHA) * p * dirr[...] + ALPHA * x
            hw = jnp.dot(h, W_ref[...], preferred_element_type=jnp.float32)
            tt = (1.0 - beta) * h + beta * hw
            tt = tt * BN_INV * g_ref[...] + b_ref[...]
            tt = jnp.maximum(tt, 0.0)
            tt = tt * dorr[...]
            oa_ref[...] = tt[:, :DH]
            ob_ref[...] = tt[:, DH:]

        one(p1a_ref, p1b_ref, dir1_ref, dor1_ref, W1_ref, g1_ref, b1_ref,
            xs1a_ref, xs1b_ref)
        one(p2a_ref, p2b_ref, dir2_ref, dor2_ref, W2_ref, g2_ref, b2_ref,
            xs2a_ref, xs2b_ref)

    col = pl.BlockSpec((RB, 1), lambda i: (i, 0))
    row = _row_block_spec(RB, D)
    half = _row_block_spec(RB, DH)
    halfshape = jax.ShapeDtypeStruct((NP, DH), jnp.float32)
    wspec = _full_spec((D, D))
    vspec = _full_spec((1, D))

    def pspec(mi, hi):
        return pl.BlockSpec((1, 1, RB, DH), lambda i: (mi, hi, i, 0))

    return pl.pallas_call(
        body,
        grid=grid,
        in_specs=[pspec(0, 0), pspec(0, 1), pspec(1, 0), pspec(1, 1),
                  col, col, col, col, row,
                  wspec, wspec, vspec, vspec, vspec, vspec],
        out_specs=[half, half, half, half],
        out_shape=[halfshape, halfshape, halfshape, halfshape],
    )(part, part, part, part, dir1, dir2, dor1, dor2, xp,
      W1, W2, g1, b1, g2, b2)


def _tc_scores(part, dir1, dir2, xp,
               W1, W2, attW, attb, attq, beta, N, NP, RB):
    D = xp.shape[1]
    DH = D // 2
    grid = (NP // RB,)

    def body(p1a_ref, p1b_ref, p2a_ref, p2b_ref,
             dir1_ref, dir2_ref, x_ref, W1_ref, W2_ref,
             attW_ref, attb_ref, attq_ref, h1_ref, h2_ref, ws_ref):
        i = pl.program_id(0)
        x = x_ref[...]

        def one(pa_ref, pb_ref, dirr, W_ref, h_ref):
            p = jnp.concatenate([pa_ref[...].reshape(RB, DH),
                                 pb_ref[...].reshape(RB, DH)], axis=1)
            h = (1.0 - ALPHA) * p * dirr[...] + ALPHA * x
            hw = jnp.dot(h, W_ref[...], preferred_element_type=jnp.float32)
            h = (1.0 - beta) * h + beta * hw
            h_ref[...] = h
            tt = jnp.tanh(jnp.dot(h, attW_ref[...],
                                  preferred_element_type=jnp.float32)
                          + attb_ref[...])
            return jnp.dot(tt, attq_ref[...],
                           preferred_element_type=jnp.float32)

        w1 = one(p1a_ref, p1b_ref, dir1_ref, W1_ref, h1_ref)
        w2 = one(p2a_ref, p2b_ref, dir2_ref, W2_ref, h2_ref)
        rowid = i * RB + lax.broadcasted_iota(jnp.int32, (RB, 1), 0)
        valid = rowid < N
        s1 = jnp.sum(jnp.where(valid, w1, 0.0))
        s2 = jnp.sum(jnp.where(valid, w2, 0.0))
        rr = lax.broadcasted_iota(jnp.int32, (8, 128), 0)
        cc = lax.broadcasted_iota(jnp.int32, (8, 128), 1)
        contrib = (jnp.where((rr == 0) & (cc == 0), s1, 0.0)
                   + jnp.where((rr == 0) & (cc == 1), s2, 0.0))

        @pl.when(i == 0)
        def _():
            ws_ref[...] = contrib

        @pl.when(i > 0)
        def _():
            ws_ref[...] = ws_ref[...] + contrib

    col = pl.BlockSpec((RB, 1), lambda i: (i, 0))
    row = _row_block_spec(RB, D)

    def pspec(mi, hi):
        return pl.BlockSpec((1, 1, RB, DH), lambda i: (mi, hi, i, 0))

    return pl.pallas_call(
        body,
        grid=grid,
        in_specs=[pspec(0, 0), pspec(0, 1), pspec(1, 0), pspec(1, 1),
                  col, col, row,
                  _full_spec((D, D)), _full_spec((D, D)),
                  _full_spec(attW.shape), _full_spec((1, attW.shape[1])),
                  _full_spec((attW.shape[1], 1))],
        out_specs=[row, row, pl.BlockSpec((8, 128), lambda i: (0, 0))],
        out_shape=[
            jax.ShapeDtypeStruct((NP, D), jnp.float32),
            jax.ShapeDtypeStruct((NP, D), jnp.float32),
            jax.ShapeDtypeStruct((8, 128), jnp.float32),
        ],
    )(part, part, part, part, dir1, dir2, xp, W1, W2, attW, attb, attq)


def _tc_combine(h1, h2, ws, N, RB):
    NP, D = h1.shape
    grid = (pl.cdiv(N, RB),)

    def body(h1_ref, h2_ref, ws_ref, o_ref):
        ws = ws_ref[...]
        rr = lax.broadcasted_iota(jnp.int32, (8, 128), 0)
        cc = lax.broadcasted_iota(jnp.int32, (8, 128), 1)
        s1 = jnp.sum(jnp.where((rr == 0) & (cc == 0), ws, 0.0))
        s2 = jnp.sum(jnp.where((rr == 0) & (cc == 1), ws, 0.0))
        m1 = s1 / N
        m2 = s2 / N
        mx = jnp.maximum(m1, m2)
        e1 = jnp.exp(m1 - mx)
        e2 = jnp.exp(m2 - mx)
        bb1 = e1 / (e1 + e2)
        bb2 = e2 / (e1 + e2)
        o_ref[...] = bb1 * h1_ref[...] + bb2 * h2_ref[...]

    row = _row_block_spec(RB, D)
    return pl.pallas_call(
        body,
        grid=grid,
        in_specs=[row, row, pl.BlockSpec((8, 128), lambda i: (0, 0))],
        out_specs=row,
        out_shape=jax.ShapeDtypeStruct((N, D), jnp.float32),
    )(h1, h2, ws)


def kernel(x, edge_index_mp1, edge_index_mp2,
           W_mp1_0, W_mp1_1, W_mp1_2, bn_g_mp1_0, bn_b_mp1_0, bn_g_mp1_1, bn_b_mp1_1,
           W_mp2_0, W_mp2_1, W_mp2_2, bn_g_mp2_0, bn_b_mp2_0, bn_g_mp2_1, bn_b_mp2_1,
           att_W, att_b, att_q):
    N, D = x.shape
    E = edge_index_mp1.shape[1]
    n_layers = 3
    RB = 1024

    NP = ((N + 1 + NSUB * LANE - 1) // (NSUB * LANE)) * (NSUB * LANE)
    CH = KB * LANE
    EPT = ((E + NSUB - 1) // NSUB + CH - 1) // CH * CH
    EP = EPT * NSUB
    ROWS_T = EPT // LANE

    def prep_edges(ei):
        pad = jnp.full((EP - E,), N, jnp.int32)
        s = jnp.concatenate([ei[0], pad]).reshape(EP // LANE, LANE)
        d = jnp.concatenate([ei[1], pad]).reshape(EP // LANE, LANE)
        return s, d

    s1, d1 = prep_edges(edge_index_mp1)
    s2, d2 = prep_edges(edge_index_mp2)

    sc_deg = _make_sc_degrees(NP, ROWS_T)
    sc_spmv = _make_sc_spmv(NP, D, ROWS_T)

    co1, ci1, co2, ci2 = sc_deg(s1, d1, s2, d2)
    co1, ci1, co2, ci2 = (v[:, None] for v in (co1, ci1, co2, ci2))

    (xs1a, xs1b, xs2a, xs2b,
     dir1, dir2, dor1, dor2) = _tc_prep(x, co1, ci1, co2, ci2, NP, RB)

    Ws1 = [W_mp1_0, W_mp1_1, W_mp1_2]
    Ws2 = [W_mp2_0, W_mp2_1, W_mp2_2]
    gs1 = [bn_g_mp1_0.reshape(1, D), bn_g_mp1_1.reshape(1, D)]
    bs1 = [bn_b_mp1_0.reshape(1, D), bn_b_mp1_1.reshape(1, D)]
    gs2 = [bn_g_mp2_0.reshape(1, D), bn_g_mp2_1.reshape(1, D)]
    bs2 = [bn_b_mp2_0.reshape(1, D), bn_b_mp2_1.reshape(1, D)]

    for l in range(n_layers - 1):
        beta = float(np.log(LAMBDA / (l + 1) + 1.0))
        part = sc_spmv(xs1a, xs1b, xs2a, xs2b, s1, d1, s2, d2)
        xs1a, xs1b, xs2a, xs2b = _tc_update(
            part, dir1, dir2, dor1, dor2, x,
            Ws1[l], Ws2[l], gs1[l], bs1[l], gs2[l], bs2[l], beta, NP, RB)

    beta = float(np.log(LAMBDA / n_layers + 1.0))
    part = sc_spmv(xs1a, xs1b, xs2a, xs2b, s1, d1, s2, d2)
    h1, h2, ws = _tc_scores(part, dir1, dir2, x,
                            Ws1[2], Ws2[2], att_W,
                            att_b.reshape(1, -1), att_q.reshape(-1, 1),
                            beta, N, NP, RB)
    return _tc_combine(h1, h2, ws, N, RB)


# speedup vs baseline: 1.3208x; 1.3208x over previous
import functools

import jax
import jax.numpy as jnp
import numpy as np
from jax import lax
from jax.experimental import pallas as pl
from jax.experimental.pallas import tpu as pltpu
from jax.experimental.pallas import tpu_sc as plsc

ALPHA = 0.2
LAMBDA = 1.0
BN_INV = float(1.0 / np.sqrt(1.0 + 1e-5))

NSUB = 16
LANE = 128
KB = 8

_SC_PARAMS = pltpu.CompilerParams(use_tc_tiling_on_sc=False)


def _fill_1d(ref, n, val):
    def body(k, _):
        ref[pl.ds(k * 16, 16)] = jnp.full((16,), val, jnp.float32)
        return _
    lax.fori_loop(0, n // 16, body, None)


def _fill_2d(ref, rows, cols, val):
    def body(r, _):
        for j in range(cols // 16):
            ref[r, pl.ds(j * 16, 16)] = jnp.full((16,), val, jnp.float32)
        return _
    lax.fori_loop(0, rows, body, None)


def _make_sc_degrees(NP, ROWS_T):
    STRIPE = NP // NSUB
    n_outer = ROWS_T // KB
    mesh = plsc.VectorSubcoreMesh(core_axis_name="c", subcore_axis_name="s")
    cnt_t = jax.ShapeDtypeStruct((NP,), jnp.float32)

    @functools.partial(
        pl.kernel,
        out_type=[cnt_t, cnt_t, cnt_t, cnt_t],
        mesh=mesh,
        compiler_params=_SC_PARAMS,
        scratch_types=[
            pltpu.VMEM((KB, LANE), jnp.int32),
            pltpu.VMEM((KB, LANE), jnp.int32),
            pltpu.VMEM((LANE,), jnp.float32),
            pltpu.VMEM((STRIPE,), jnp.float32),
            pltpu.VMEM_SHARED((NP,), jnp.float32),
            pltpu.VMEM_SHARED((NP,), jnp.float32),
            pltpu.SemaphoreType.DMA,
        ],
    )
    def k(s1_h, d1_h, s2_h, d2_h, co1_h, ci1_h, co2_h, ci2_h,
          sidx_v, didx_v, ones_v, zb_v, acc_o, acc_i, dsem):
        c = lax.axis_index("c")
        t = lax.axis_index("s")
        _fill_1d(ones_v, LANE, 1.0)
        _fill_1d(zb_v, STRIPE, 0.0)
        pltpu.sync_copy(zb_v, acc_o.at[pl.ds(STRIPE * t, STRIPE)])
        pltpu.sync_copy(zb_v, acc_i.at[pl.ds(STRIPE * t, STRIPE)])
        plsc.subcore_barrier()

        def run(s_h, d_h, co_h, ci_h):
            def outer(i, carry):
                rb = t * ROWS_T + i * KB
                pltpu.sync_copy(s_h.at[pl.ds(rb, KB)], sidx_v)
                pltpu.sync_copy(d_h.at[pl.ds(rb, KB)], didx_v)
                for j in range(KB):
                    pltpu.async_copy(ones_v, acc_o.at[sidx_v.at[j]], dsem,
                                     add=True)
                for j in range(KB):
                    pltpu.async_copy(ones_v, acc_i.at[didx_v.at[j]], dsem,
                                     add=True)
                for j in range(KB):
                    pltpu.make_async_copy(
                        ones_v, acc_o.at[sidx_v.at[j]], dsem).wait()
                    pltpu.make_async_copy(
                        ones_v, acc_i.at[didx_v.at[j]], dsem).wait()
                return carry
            lax.fori_loop(0, n_outer, outer, 0)
            plsc.subcore_barrier()
            pltpu.sync_copy(acc_o.at[pl.ds(STRIPE * t, STRIPE)],
                            co_h.at[pl.ds(STRIPE * t, STRIPE)])
            pltpu.sync_copy(acc_i.at[pl.ds(STRIPE * t, STRIPE)],
                            ci_h.at[pl.ds(STRIPE * t, STRIPE)])

        @pl.when(c == 0)
        def _():
            run(s1_h, d1_h, co1_h, ci1_h)

        @pl.when(c == 1)
        def _():
            run(s2_h, d2_h, co2_h, ci2_h)

    return k


def _make_sc_spmv(NP, D, ROWS_T):
    DH = D // 2
    STRIPE = NP // NSUB
    PH = 4
    PROWS = ROWS_T // PH
    NGRP = PROWS // 4
    mesh = plsc.VectorSubcoreMesh(core_axis_name="c", subcore_axis_name="s")

    @functools.partial(
        pl.kernel,
        out_type=jax.ShapeDtypeStruct((2, 2, NP, DH), jnp.float32),
        mesh=mesh,
        compiler_params=_SC_PARAMS,
        scratch_types=(
            [pltpu.VMEM((PROWS, LANE), jnp.int32),
             pltpu.VMEM((PROWS, LANE), jnp.int32),
             pltpu.VMEM((4 * LANE, DH), jnp.float32),
             pltpu.VMEM_SHARED((NP, DH), jnp.float32),
             pltpu.VMEM_SHARED((NP, DH), jnp.float32)]
            + [pltpu.SemaphoreType.DMA] * 8
        ),
    )
    def k(xs1a_h, xs1b_h, xs2a_h, xs2b_h, s1_h, d1_h, s2_h, d2_h, out_h,
          sidx_v, didx_v, rows_v, sxs, sacc,
          g0, g1, g2, g3, ss0, ss1, ss2, ss3):
        c = lax.axis_index("c")
        t = lax.axis_index("s")
        gsem = [g0, g1, g2, g3]
        ssem = [ss0, ss1, ss2, ss3]

        def slot(k_):
            return rows_v.at[pl.ds(k_ * LANE, LANE)]

        def gi(r, k_):
            pltpu.async_copy(sxs.at[sidx_v.at[r]], slot(k_), gsem[k_])

        def gw(r, k_):
            pltpu.make_async_copy(sxs.at[sidx_v.at[r]], slot(k_),
                                  gsem[k_]).wait()

        def si(r, k_):
            pltpu.async_copy(slot(k_), sacc.at[didx_v.at[r]], ssem[k_],
                             add=True)

        def sw(r, k_):
            pltpu.make_async_copy(slot(k_), sacc.at[didx_v.at[r]],
                                  ssem[k_]).wait()

        def run(xsa_h, xsb_h, s_h, d_h, mi):
            for half, xs_h in ((0, xsa_h), (1, xsb_h)):
                pltpu.sync_copy(xs_h.at[pl.ds(STRIPE * t, STRIPE)],
                                sxs.at[pl.ds(STRIPE * t, STRIPE)])
                _fill_2d(rows_v, LANE, DH, 0.0)
                for q in range(STRIPE // LANE):
                    pltpu.sync_copy(
                        rows_v.at[pl.ds(0, LANE)],
                        sacc.at[pl.ds(STRIPE * t + LANE * q, LANE)])
                plsc.subcore_barrier()

                def phase(ph, carry):
                    rb = t * ROWS_T + ph * PROWS
                    pltpu.sync_copy(s_h.at[pl.ds(rb, PROWS)], sidx_v)
                    pltpu.sync_copy(d_h.at[pl.ds(rb, PROWS)], didx_v)
                    gi(0, 0)
                    gi(1, 1)
                    gi(2, 2)
                    gw(0, 0)
                    si(0, 0)
                    gi(3, 3)
                    gw(1, 1)
                    si(1, 1)

                    def group(g, cr):
                        for k_ in range(4):
                            r = g * 4 + k_
                            sw(r - 4, k_)
                            gi(r, k_)
                            k2 = (k_ + 2) % 4
                            gw(r - 2, k2)
                            si(r - 2, k2)
                        return cr
                    lax.fori_loop(1, NGRP, group, 0)
                    gw(PROWS - 2, 2)
                    si(PROWS - 2, 2)
                    gw(PROWS - 1, 3)
                    si(PROWS - 1, 3)
                    for k_ in range(4):
                        sw(0, k_)
                    return carry
                lax.fori_loop(0, PH, phase, 0)
                plsc.subcore_barrier()
                pltpu.sync_copy(sacc.at[pl.ds(STRIPE * t, STRIPE)],
                                out_h.at[mi, half, pl.ds(STRIPE * t, STRIPE)])

        @pl.when(c == 0)
        def _():
            run(xs1a_h, xs1b_h, s1_h, d1_h, 0)

        @pl.when(c == 1)
        def _():
            run(xs2a_h, xs2b_h, s2_h, d2_h, 1)

    return k


def _row_block_spec(RB, D):
    return pl.BlockSpec((RB, D), lambda i: (i, 0))


def _full_spec(shape):
    return pl.BlockSpec(shape, lambda i: tuple(0 for _ in shape))


def _tc_prep(xp, co1, ci1, co2, ci2, NP, RB):
    D = xp.shape[1]
    grid = (NP // RB,)

    DH = D // 2

    def body(x_ref, co1_ref, ci1_ref, co2_ref, ci2_ref,
             xs1a_ref, xs1b_ref, xs2a_ref, xs2b_ref,
             dir1_ref, dir2_ref, dor1_ref, dor2_ref):
        x = x_ref[...]
        dor1 = lax.rsqrt(jnp.maximum(co1_ref[...], 1.0))
        dor2 = lax.rsqrt(jnp.maximum(co2_ref[...], 1.0))
        dir1_ref[...] = lax.rsqrt(jnp.maximum(ci1_ref[...], 1.0))
        dir2_ref[...] = lax.rsqrt(jnp.maximum(ci2_ref[...], 1.0))
        dor1_ref[...] = dor1
        dor2_ref[...] = dor2
        xs1 = x * dor1
        xs2 = x * dor2
        xs1a_ref[...] = xs1[:, :DH]
        xs1b_ref[...] = xs1[:, DH:]
        xs2a_ref[...] = xs2[:, :DH]
        xs2b_ref[...] = xs2[:, DH:]

    col = pl.BlockSpec((RB, 1), lambda i: (i, 0))
    halfspec = _row_block_spec(RB, DH)
    halfshape = jax.ShapeDtypeStruct((NP, DH), jnp.float32)
    return pl.pallas_call(
        body,
        grid=grid,
        in_specs=[_row_block_spec(RB, D), col, col, col, col],
        out_specs=[halfspec, halfspec, halfspec, halfspec,
                   col, col, col, col],
        out_shape=[
            halfshape, halfshape, halfshape, halfshape,
            jax.ShapeDtypeStruct((NP, 1), jnp.float32),
            jax.ShapeDtypeStruct((NP, 1), jnp.float32),
            jax.ShapeDtypeStruct((NP, 1), jnp.float32),
            jax.ShapeDtypeStruct((NP, 1), jnp.float32),
        ],
    )(xp, co1, ci1, co2, ci2)


def _tc_update(part, dir1, dir2, dor1, dor2, xp,
               W1, W2, g1, b1, g2, b2, beta, NP, RB):
    D = xp.shape[1]
    DH = D // 2
    grid = (NP // RB,)

    def body(p1a_ref, p1b_ref, p2a_ref, p2b_ref,
             dir1_ref, dir2_ref, dor1_ref, dor2_ref, x_ref,
             W1_ref, W2_ref, g1_ref, b1_ref, g2_ref, b2_ref,
             xs1a_ref, xs1b_ref, xs2a_ref, xs2b_ref):
        x = x_ref[...]

        def one(pa_ref, pb_ref, dirr, dorr, W_ref, g_ref, b_ref,
                oa_ref, ob_ref):
            p = jnp.concatenate([pa_ref[...].reshape(RB, DH),
                                 pb_ref[...].reshape(RB, DH)], axis=1)
            h = (1.0 - ALPHA) * p * dirr[...] + ALPHA * x
            hw = jnp.dot(h, W_ref[...], preferred_element_type=jnp.float32)
            tt = (1.0 - beta) * h + beta * hw
            tt = tt * BN_INV * g_ref[...] + b_ref[...]
            tt = jnp.maximum(tt, 0.0)
            tt = tt * dorr[...]
            oa_ref[...] = tt[:, :DH]
            ob_ref[...] = tt[:, DH:]

        one(p1a_ref, p1b_ref, dir1_ref, dor1_ref, W1_ref, g1_ref, b1_ref,
            xs1a_ref, xs1b_ref)
        one(p2a_ref, p2b_ref, dir2_ref, dor2_ref, W2_ref, g2_ref, b2_ref,
            xs2a_ref, xs2b_ref)

    col = pl.BlockSpec((RB, 1), lambda i: (i, 0))
    row = _row_block_spec(RB, D)
    half = _row_block_spec(RB, DH)
    halfshape = jax.ShapeDtypeStruct((NP, DH), jnp.float32)
    wspec = _full_spec((D, D))
    vspec = _full_spec((1, D))

    def pspec(mi, hi):
        return pl.BlockSpec((1, 1, RB, DH), lambda i: (mi, hi, i, 0))

    return pl.pallas_call(
        body,
        grid=grid,
        in_specs=[pspec(0, 0), pspec(0, 1), pspec(1, 0), pspec(1, 1),
                  col, col, col, col, row,
                  wspec, wspec, vspec, vspec, vspec, vspec],
        out_specs=[half, half, half, half],
        out_shape=[halfshape, halfshape, halfshape, halfshape],
    )(part, part, part, part, dir1, dir2, dor1, dor2, xp,
      W1, W2, g1, b1, g2, b2)


def _tc_scores(part, dir1, dir2, xp,
               W1, W2, attW, attb, attq, beta, N, NP, RB):
    D = xp.shape[1]
    DH = D // 2
    grid = (NP // RB,)

    def body(p1a_ref, p1b_ref, p2a_ref, p2b_ref,
             dir1_ref, dir2_ref, x_ref, W1_ref, W2_ref,
             attW_ref, attb_ref, attq_ref, h1_ref, h2_ref, ws_ref):
        i = pl.program_id(0)
        x = x_ref[...]

        def one(pa_ref, pb_ref, dirr, W_ref, h_ref):
            p = jnp.concatenate([pa_ref[...].reshape(RB, DH),
                                 pb_ref[...].reshape(RB, DH)], axis=1)
            h = (1.0 - ALPHA) * p * dirr[...] + ALPHA * x
            hw = jnp.dot(h, W_ref[...], preferred_element_type=jnp.float32)
            h = (1.0 - beta) * h + beta * hw
            h_ref[...] = h
            tt = jnp.tanh(jnp.dot(h, attW_ref[...],
                                  preferred_element_type=jnp.float32)
                          + attb_ref[...])
            return jnp.dot(tt, attq_ref[...],
                           preferred_element_type=jnp.float32)

        w1 = one(p1a_ref, p1b_ref, dir1_ref, W1_ref, h1_ref)
        w2 = one(p2a_ref, p2b_ref, dir2_ref, W2_ref, h2_ref)
        rowid = i * RB + lax.broadcasted_iota(jnp.int32, (RB, 1), 0)
        valid = rowid < N
        s1 = jnp.sum(jnp.where(valid, w1, 0.0))
        s2 = jnp.sum(jnp.where(valid, w2, 0.0))
        rr = lax.broadcasted_iota(jnp.int32, (8, 128), 0)
        cc = lax.broadcasted_iota(jnp.int32, (8, 128), 1)
        contrib = (jnp.where((rr == 0) & (cc == 0), s1, 0.0)
                   + jnp.where((rr == 0) & (cc == 1), s2, 0.0))

        @pl.when(i == 0)
        def _():
            ws_ref[...] = contrib

        @pl.when(i > 0)
        def _():
            ws_ref[...] = ws_ref[...] + contrib

    col = pl.BlockSpec((RB, 1), lambda i: (i, 0))
    row = _row_block_spec(RB, D)

    def pspec(mi, hi):
        return pl.BlockSpec((1, 1, RB, DH), lambda i: (mi, hi, i, 0))

    return pl.pallas_call(
        body,
        grid=grid,
        in_specs=[pspec(0, 0), pspec(0, 1), pspec(1, 0), pspec(1, 1),
                  col, col, row,
                  _full_spec((D, D)), _full_spec((D, D)),
                  _full_spec(attW.shape), _full_spec((1, attW.shape[1])),
                  _full_spec((attW.shape[1], 1))],
        out_specs=[row, row, pl.BlockSpec((8, 128), lambda i: (0, 0))],
        out_shape=[
            jax.ShapeDtypeStruct((NP, D), jnp.float32),
            jax.ShapeDtypeStruct((NP, D), jnp.float32),
            jax.ShapeDtypeStruct((8, 128), jnp.float32),
        ],
    )(part, part, part, part, dir1, dir2, xp, W1, W2, attW, attb, attq)


def _tc_combine(h1, h2, ws, N, RB):
    NP, D = h1.shape
    grid = (pl.cdiv(N, RB),)

    def body(h1_ref, h2_ref, ws_ref, o_ref):
        ws = ws_ref[...]
        rr = lax.broadcasted_iota(jnp.int32, (8, 128), 0)
        cc = lax.broadcasted_iota(jnp.int32, (8, 128), 1)
        s1 = jnp.sum(jnp.where((rr == 0) & (cc == 0), ws, 0.0))
        s2 = jnp.sum(jnp.where((rr == 0) & (cc == 1), ws, 0.0))
        m1 = s1 / N
        m2 = s2 / N
        mx = jnp.maximum(m1, m2)
        e1 = jnp.exp(m1 - mx)
        e2 = jnp.exp(m2 - mx)
        bb1 = e1 / (e1 + e2)
        bb2 = e2 / (e1 + e2)
        o_ref[...] = bb1 * h1_ref[...] + bb2 * h2_ref[...]

    row = _row_block_spec(RB, D)
    return pl.pallas_call(
        body,
        grid=grid,
        in_specs=[row, row, pl.BlockSpec((8, 128), lambda i: (0, 0))],
        out_specs=row,
        out_shape=jax.ShapeDtypeStruct((N, D), jnp.float32),
    )(h1, h2, ws)


def kernel(x, edge_index_mp1, edge_index_mp2,
           W_mp1_0, W_mp1_1, W_mp1_2, bn_g_mp1_0, bn_b_mp1_0, bn_g_mp1_1, bn_b_mp1_1,
           W_mp2_0, W_mp2_1, W_mp2_2, bn_g_mp2_0, bn_b_mp2_0, bn_g_mp2_1, bn_b_mp2_1,
           att_W, att_b, att_q):
    N, D = x.shape
    E = edge_index_mp1.shape[1]
    n_layers = 3
    RB = 1024

    NP = ((N + 1 + NSUB * LANE - 1) // (NSUB * LANE)) * (NSUB * LANE)
    CH = KB * LANE
    EPT = ((E + NSUB - 1) // NSUB + CH - 1) // CH * CH
    EP = EPT * NSUB
    ROWS_T = EPT // LANE

    def prep_edges(ei):
        pad = jnp.full((EP - E,), N, jnp.int32)
        s = jnp.concatenate([ei[0], pad]).reshape(EP // LANE, LANE)
        d = jnp.concatenate([ei[1], pad]).reshape(EP // LANE, LANE)
        return s, d

    s1, d1 = prep_edges(edge_index_mp1)
    s2, d2 = prep_edges(edge_index_mp2)

    sc_deg = _make_sc_degrees(NP, ROWS_T)
    sc_spmv = _make_sc_spmv(NP, D, ROWS_T)

    co1, ci1, co2, ci2 = sc_deg(s1, d1, s2, d2)
    co1, ci1, co2, ci2 = (v[:, None] for v in (co1, ci1, co2, ci2))

    (xs1a, xs1b, xs2a, xs2b,
     dir1, dir2, dor1, dor2) = _tc_prep(x, co1, ci1, co2, ci2, NP, RB)

    Ws1 = [W_mp1_0, W_mp1_1, W_mp1_2]
    Ws2 = [W_mp2_0, W_mp2_1, W_mp2_2]
    gs1 = [bn_g_mp1_0.reshape(1, D), bn_g_mp1_1.reshape(1, D)]
    bs1 = [bn_b_mp1_0.reshape(1, D), bn_b_mp1_1.reshape(1, D)]
    gs2 = [bn_g_mp2_0.reshape(1, D), bn_g_mp2_1.reshape(1, D)]
    bs2 = [bn_b_mp2_0.reshape(1, D), bn_b_mp2_1.reshape(1, D)]

    for l in range(n_layers - 1):
        beta = float(np.log(LAMBDA / (l + 1) + 1.0))
        part = sc_spmv(xs1a, xs1b, xs2a, xs2b, s1, d1, s2, d2)
        xs1a, xs1b, xs2a, xs2b = _tc_update(
            part, dir1, dir2, dor1, dor2, x,
            Ws1[l], Ws2[l], gs1[l], bs1[l], gs2[l], bs2[l], beta, NP, RB)

    beta = float(np.log(LAMBDA / n_layers + 1.0))
    part = sc_spmv(xs1a, xs1b, xs2a, xs2b, s1, d1, s2, d2)
    h1, h2, ws = _tc_scores(part, dir1, dir2, x,
                            Ws1[2], Ws2[2], att_W,
                            att_b.reshape(1, -1), att_q.reshape(-1, 1),
                            beta, N, NP, RB)
    return _tc_combine(h1, h2, ws, N, RB)


# speedup vs baseline: 1.4649x; 1.1091x over previous
import functools

import jax
import jax.numpy as jnp
import numpy as np
from jax import lax
from jax.experimental import pallas as pl
from jax.experimental.pallas import tpu as pltpu
from jax.experimental.pallas import tpu_sc as plsc

ALPHA = 0.2
LAMBDA = 1.0
BN_INV = float(1.0 / np.sqrt(1.0 + 1e-5))

NSUB = 16
LANE = 128
KB = 8

_SC_PARAMS = pltpu.CompilerParams(use_tc_tiling_on_sc=False)


def _fill_1d(ref, n, val):
    def body(k, _):
        ref[pl.ds(k * 16, 16)] = jnp.full((16,), val, jnp.float32)
        return _
    lax.fori_loop(0, n // 16, body, None)


def _fill_2d(ref, rows, cols, val):
    def body(r, _):
        for j in range(cols // 16):
            ref[r, pl.ds(j * 16, 16)] = jnp.full((16,), val, jnp.float32)
        return _
    lax.fori_loop(0, rows, body, None)


def _make_sc_degrees(NP, ROWS_T):
    STRIPE = NP // NSUB
    n_outer = ROWS_T // KB
    mesh = plsc.VectorSubcoreMesh(core_axis_name="c", subcore_axis_name="s")
    cnt_t = jax.ShapeDtypeStruct((NP,), jnp.float32)

    @functools.partial(
        pl.kernel,
        out_type=[cnt_t, cnt_t, cnt_t, cnt_t],
        mesh=mesh,
        compiler_params=_SC_PARAMS,
        scratch_types=[
            pltpu.VMEM((KB, LANE), jnp.int32),
            pltpu.VMEM((KB, LANE), jnp.int32),
            pltpu.VMEM((LANE,), jnp.float32),
            pltpu.VMEM((STRIPE,), jnp.float32),
            pltpu.VMEM_SHARED((NP,), jnp.float32),
            pltpu.VMEM_SHARED((NP,), jnp.float32),
            pltpu.SemaphoreType.DMA,
        ],
    )
    def k(s1_h, d1_h, s2_h, d2_h, co1_h, ci1_h, co2_h, ci2_h,
          sidx_v, didx_v, ones_v, zb_v, acc_o, acc_i, dsem):
        c = lax.axis_index("c")
        t = lax.axis_index("s")
        _fill_1d(ones_v, LANE, 1.0)
        _fill_1d(zb_v, STRIPE, 0.0)
        pltpu.sync_copy(zb_v, acc_o.at[pl.ds(STRIPE * t, STRIPE)])
        pltpu.sync_copy(zb_v, acc_i.at[pl.ds(STRIPE * t, STRIPE)])
        plsc.subcore_barrier()

        def run(s_h, d_h, co_h, ci_h):
            def outer(i, carry):
                rb = t * ROWS_T + i * KB
                pltpu.sync_copy(s_h.at[pl.ds(rb, KB)], sidx_v)
                pltpu.sync_copy(d_h.at[pl.ds(rb, KB)], didx_v)
                for j in range(KB):
                    pltpu.async_copy(ones_v, acc_o.at[sidx_v.at[j]], dsem,
                                     add=True)
                for j in range(KB):
                    pltpu.async_copy(ones_v, acc_i.at[didx_v.at[j]], dsem,
                                     add=True)
                for j in range(KB):
                    pltpu.make_async_copy(
                        ones_v, acc_o.at[sidx_v.at[j]], dsem).wait()
                    pltpu.make_async_copy(
                        ones_v, acc_i.at[didx_v.at[j]], dsem).wait()
                return carry
            lax.fori_loop(0, n_outer, outer, 0)
            plsc.subcore_barrier()
            pltpu.sync_copy(acc_o.at[pl.ds(STRIPE * t, STRIPE)],
                            co_h.at[pl.ds(STRIPE * t, STRIPE)])
            pltpu.sync_copy(acc_i.at[pl.ds(STRIPE * t, STRIPE)],
                            ci_h.at[pl.ds(STRIPE * t, STRIPE)])

        @pl.when(c == 0)
        def _():
            run(s1_h, d1_h, co1_h, ci1_h)

        @pl.when(c == 1)
        def _():
            run(s2_h, d2_h, co2_h, ci2_h)

    return k


def _make_sc_spmv(NP, D, ROWS_T):
    DH = D // 2
    STRIPE = NP // NSUB
    PH = 4
    PROWS = ROWS_T // PH
    NGRP = PROWS // 4
    mesh = plsc.VectorSubcoreMesh(core_axis_name="c", subcore_axis_name="s")

    @functools.partial(
        pl.kernel,
        out_type=jax.ShapeDtypeStruct((2, NP, DH), jnp.float32),
        mesh=mesh,
        compiler_params=_SC_PARAMS,
        scratch_types=(
            [pltpu.VMEM((PROWS, LANE), jnp.int32),
             pltpu.VMEM((PROWS, LANE), jnp.int32),
             pltpu.VMEM((4 * LANE, DH), jnp.float32),
             pltpu.VMEM_SHARED((NP, DH), jnp.float32),
             pltpu.VMEM_SHARED((NP, DH), jnp.float32)]
            + [pltpu.SemaphoreType.DMA] * 8
        ),
    )
    def k(xsa_h, xsb_h, s_h, d_h, out_h,
          sidx_v, didx_v, rows_v, sxs, sacc,
          g0, g1, g2, g3, ss0, ss1, ss2, ss3):
        c = lax.axis_index("c")
        t = lax.axis_index("s")
        gsem = [g0, g1, g2, g3]
        ssem = [ss0, ss1, ss2, ss3]

        def slot(k_):
            return rows_v.at[pl.ds(k_ * LANE, LANE)]

        def gi(r, k_):
            pltpu.async_copy(sxs.at[sidx_v.at[r]], slot(k_), gsem[k_])

        def gw(r, k_):
            pltpu.make_async_copy(sxs.at[sidx_v.at[r]], slot(k_),
                                  gsem[k_]).wait()

        def si(r, k_):
            pltpu.async_copy(slot(k_), sacc.at[didx_v.at[r]], ssem[k_],
                             add=True)

        def sw(r, k_):
            pltpu.make_async_copy(slot(k_), sacc.at[didx_v.at[r]],
                                  ssem[k_]).wait()

        def run(xs_h, half):
            if True:
                pltpu.sync_copy(xs_h.at[pl.ds(STRIPE * t, STRIPE)],
                                sxs.at[pl.ds(STRIPE * t, STRIPE)])
                _fill_2d(rows_v, LANE, DH, 0.0)
                for q in range(STRIPE // LANE):
                    pltpu.sync_copy(
                        rows_v.at[pl.ds(0, LANE)],
                        sacc.at[pl.ds(STRIPE * t + LANE * q, LANE)])
                plsc.subcore_barrier()

                def phase(ph, carry):
                    rb = t * ROWS_T + ph * PROWS
                    pltpu.sync_copy(s_h.at[pl.ds(rb, PROWS)], sidx_v)
                    pltpu.sync_copy(d_h.at[pl.ds(rb, PROWS)], didx_v)
                    gi(0, 0)
                    gi(1, 1)
                    gi(2, 2)
                    gw(0, 0)
                    si(0, 0)
                    gi(3, 3)
                    gw(1, 1)
                    si(1, 1)

                    def group(g, cr):
                        for k_ in range(4):
                            r = g * 4 + k_
                            sw(r - 4, k_)
                            gi(r, k_)
                            k2 = (k_ + 2) % 4
                            gw(r - 2, k2)
                            si(r - 2, k2)
                        return cr
                    lax.fori_loop(1, NGRP, group, 0)
                    gw(PROWS - 2, 2)
                    si(PROWS - 2, 2)
                    gw(PROWS - 1, 3)
                    si(PROWS - 1, 3)
                    for k_ in range(4):
                        sw(0, k_)
                    return carry
                lax.fori_loop(0, PH, phase, 0)
                plsc.subcore_barrier()
                pltpu.sync_copy(sacc.at[pl.ds(STRIPE * t, STRIPE)],
                                out_h.at[half, pl.ds(STRIPE * t, STRIPE)])

        @pl.when(c == 0)
        def _():
            run(xsa_h, 0)

        @pl.when(c == 1)
        def _():
            run(xsb_h, 1)

    return k


def _row_block_spec(RB, D):
    return pl.BlockSpec((RB, D), lambda i: (i, 0))


def _full_spec(shape):
    return pl.BlockSpec(shape, lambda i: tuple(0 for _ in shape))


def _tc_prep(xp, co1, ci1, co2, ci2, NP, RB):
    D = xp.shape[1]
    grid = (NP // RB,)

    DH = D // 2

    def body(x_ref, co1_ref, ci1_ref, co2_ref, ci2_ref,
             xs1a_ref, xs1b_ref, xs2a_ref, xs2b_ref,
             dir1_ref, dir2_ref, dor1_ref, dor2_ref):
        x = x_ref[...]
        dor1 = lax.rsqrt(jnp.maximum(co1_ref[...], 1.0))
        dor2 = lax.rsqrt(jnp.maximum(co2_ref[...], 1.0))
        dir1_ref[...] = lax.rsqrt(jnp.maximum(ci1_ref[...], 1.0))
        dir2_ref[...] = lax.rsqrt(jnp.maximum(ci2_ref[...], 1.0))
        dor1_ref[...] = dor1
        dor2_ref[...] = dor2
        xs1 = x * dor1
        xs2 = x * dor2
        xs1a_ref[...] = xs1[:, :DH]
        xs1b_ref[...] = xs1[:, DH:]
        xs2a_ref[...] = xs2[:, :DH]
        xs2b_ref[...] = xs2[:, DH:]

    col = pl.BlockSpec((RB, 1), lambda i: (i, 0))
    halfspec = _row_block_spec(RB, DH)
    halfshape = jax.ShapeDtypeStruct((NP, DH), jnp.float32)
    return pl.pallas_call(
        body,
        grid=grid,
        in_specs=[_row_block_spec(RB, D), col, col, col, col],
        out_specs=[halfspec, halfspec, halfspec, halfspec,
                   col, col, col, col],
        out_shape=[
            halfshape, halfshape, halfshape, halfshape,
            jax.ShapeDtypeStruct((NP, 1), jnp.float32),
            jax.ShapeDtypeStruct((NP, 1), jnp.float32),
            jax.ShapeDtypeStruct((NP, 1), jnp.float32),
            jax.ShapeDtypeStruct((NP, 1), jnp.float32),
        ],
    )(xp, co1, ci1, co2, ci2)


def _tc_update(part, dirr, dorr, xp, W, g, b, beta, NP, RB):
    D = xp.shape[1]
    DH = D // 2
    grid = (NP // RB,)

    def body(pa_ref, pb_ref, dir_ref, dor_ref, x_ref,
             W_ref, g_ref, b_ref, oa_ref, ob_ref):
        x = x_ref[...]
        p = jnp.concatenate([pa_ref[...].reshape(RB, DH),
                             pb_ref[...].reshape(RB, DH)], axis=1)
        h = (1.0 - ALPHA) * p * dir_ref[...] + ALPHA * x
        hw = jnp.dot(h, W_ref[...], preferred_element_type=jnp.float32)
        tt = (1.0 - beta) * h + beta * hw
        tt = tt * BN_INV * g_ref[...] + b_ref[...]
        tt = jnp.maximum(tt, 0.0)
        tt = tt * dor_ref[...]
        oa_ref[...] = tt[:, :DH]
        ob_ref[...] = tt[:, DH:]

    col = pl.BlockSpec((RB, 1), lambda i: (i, 0))
    row = _row_block_spec(RB, D)
    half = _row_block_spec(RB, DH)
    halfshape = jax.ShapeDtypeStruct((NP, DH), jnp.float32)

    def pspec(hi):
        return pl.BlockSpec((1, RB, DH), lambda i: (hi, i, 0))

    return pl.pallas_call(
        body,
        grid=grid,
        in_specs=[pspec(0), pspec(1), col, col, row,
                  _full_spec((D, D)), _full_spec((1, D)), _full_spec((1, D))],
        out_specs=[half, half],
        out_shape=[halfshape, halfshape],
    )(part, part, dirr, dorr, xp, W, g, b)


def _tc_scores(part1, part2, dir1, dir2, xp,
               W1, W2, attW, attb, attq, beta, N, NP, RB):
    D = xp.shape[1]
    DH = D // 2
    grid = (NP // RB,)

    def body(p1a_ref, p1b_ref, p2a_ref, p2b_ref,
             dir1_ref, dir2_ref, x_ref, W1_ref, W2_ref,
             attW_ref, attb_ref, attq_ref, h1_ref, h2_ref, ws_ref):
        i = pl.program_id(0)
        x = x_ref[...]

        def one(pa_ref, pb_ref, dirr, W_ref, h_ref):
            p = jnp.concatenate([pa_ref[...].reshape(RB, DH),
                                 pb_ref[...].reshape(RB, DH)], axis=1)
            h = (1.0 - ALPHA) * p * dirr[...] + ALPHA * x
            hw = jnp.dot(h, W_ref[...], preferred_element_type=jnp.float32)
            h = (1.0 - beta) * h + beta * hw
            h_ref[...] = h
            tt = jnp.tanh(jnp.dot(h, attW_ref[...],
                                  preferred_element_type=jnp.float32)
                          + attb_ref[...])
            return jnp.dot(tt, attq_ref[...],
                           preferred_element_type=jnp.float32)

        w1 = one(p1a_ref, p1b_ref, dir1_ref, W1_ref, h1_ref)
        w2 = one(p2a_ref, p2b_ref, dir2_ref, W2_ref, h2_ref)
        rowid = i * RB + lax.broadcasted_iota(jnp.int32, (RB, 1), 0)
        valid = rowid < N
        s1 = jnp.sum(jnp.where(valid, w1, 0.0))
        s2 = jnp.sum(jnp.where(valid, w2, 0.0))
        rr = lax.broadcasted_iota(jnp.int32, (8, 128), 0)
        cc = lax.broadcasted_iota(jnp.int32, (8, 128), 1)
        contrib = (jnp.where((rr == 0) & (cc == 0), s1, 0.0)
                   + jnp.where((rr == 0) & (cc == 1), s2, 0.0))

        @pl.when(i == 0)
        def _():
            ws_ref[...] = contrib

        @pl.when(i > 0)
        def _():
            ws_ref[...] = ws_ref[...] + contrib

    col = pl.BlockSpec((RB, 1), lambda i: (i, 0))
    row = _row_block_spec(RB, D)

    def pspec(hi):
        return pl.BlockSpec((1, RB, DH), lambda i: (hi, i, 0))

    return pl.pallas_call(
        body,
        grid=grid,
        in_specs=[pspec(0), pspec(1), pspec(0), pspec(1),
                  col, col, row,
                  _full_spec((D, D)), _full_spec((D, D)),
                  _full_spec(attW.shape), _full_spec((1, attW.shape[1])),
                  _full_spec((attW.shape[1], 1))],
        out_specs=[row, row, pl.BlockSpec((8, 128), lambda i: (0, 0))],
        out_shape=[
            jax.ShapeDtypeStruct((NP, D), jnp.float32),
            jax.ShapeDtypeStruct((NP, D), jnp.float32),
            jax.ShapeDtypeStruct((8, 128), jnp.float32),
        ],
    )(part1, part1, part2, part2, dir1, dir2, xp, W1, W2, attW, attb, attq)


def _tc_combine(h1, h2, ws, N, RB):
    NP, D = h1.shape
    grid = (pl.cdiv(N, RB),)

    def body(h1_ref, h2_ref, ws_ref, o_ref):
        ws = ws_ref[...]
        rr = lax.broadcasted_iota(jnp.int32, (8, 128), 0)
        cc = lax.broadcasted_iota(jnp.int32, (8, 128), 1)
        s1 = jnp.sum(jnp.where((rr == 0) & (cc == 0), ws, 0.0))
        s2 = jnp.sum(jnp.where((rr == 0) & (cc == 1), ws, 0.0))
        m1 = s1 / N
        m2 = s2 / N
        mx = jnp.maximum(m1, m2)
        e1 = jnp.exp(m1 - mx)
        e2 = jnp.exp(m2 - mx)
        bb1 = e1 / (e1 + e2)
        bb2 = e2 / (e1 + e2)
        o_ref[...] = bb1 * h1_ref[...] + bb2 * h2_ref[...]

    row = _row_block_spec(RB, D)
    return pl.pallas_call(
        body,
        grid=grid,
        in_specs=[row, row, pl.BlockSpec((8, 128), lambda i: (0, 0))],
        out_specs=row,
        out_shape=jax.ShapeDtypeStruct((N, D), jnp.float32),
    )(h1, h2, ws)


def kernel(x, edge_index_mp1, edge_index_mp2,
           W_mp1_0, W_mp1_1, W_mp1_2, bn_g_mp1_0, bn_b_mp1_0, bn_g_mp1_1, bn_b_mp1_1,
           W_mp2_0, W_mp2_1, W_mp2_2, bn_g_mp2_0, bn_b_mp2_0, bn_g_mp2_1, bn_b_mp2_1,
           att_W, att_b, att_q):
    N, D = x.shape
    E = edge_index_mp1.shape[1]
    n_layers = 3
    RB = 1024

    NP = ((N + 1 + NSUB * LANE - 1) // (NSUB * LANE)) * (NSUB * LANE)
    CH = KB * LANE
    EPT = ((E + NSUB - 1) // NSUB + CH - 1) // CH * CH
    EP = EPT * NSUB
    ROWS_T = EPT // LANE

    def prep_edges(ei):
        pad = jnp.full((EP - E,), N, jnp.int32)
        s = jnp.concatenate([ei[0], pad]).reshape(EP // LANE, LANE)
        d = jnp.concatenate([ei[1], pad]).reshape(EP // LANE, LANE)
        return s, d

    s1, d1 = prep_edges(edge_index_mp1)
    s2, d2 = prep_edges(edge_index_mp2)

    sc_deg = _make_sc_degrees(NP, ROWS_T)
    sc_spmv = _make_sc_spmv(NP, D, ROWS_T)

    co1, ci1, co2, ci2 = sc_deg(s1, d1, s2, d2)
    co1, ci1, co2, ci2 = (v[:, None] for v in (co1, ci1, co2, ci2))

    (xs1a, xs1b, xs2a, xs2b,
     dir1, dir2, dor1, dor2) = _tc_prep(x, co1, ci1, co2, ci2, NP, RB)

    Ws1 = [W_mp1_0, W_mp1_1, W_mp1_2]
    Ws2 = [W_mp2_0, W_mp2_1, W_mp2_2]
    gs1 = [bn_g_mp1_0.reshape(1, D), bn_g_mp1_1.reshape(1, D)]
    bs1 = [bn_b_mp1_0.reshape(1, D), bn_b_mp1_1.reshape(1, D)]
    gs2 = [bn_g_mp2_0.reshape(1, D), bn_g_mp2_1.reshape(1, D)]
    bs2 = [bn_b_mp2_0.reshape(1, D), bn_b_mp2_1.reshape(1, D)]

    for l in range(n_layers - 1):
        beta = float(np.log(LAMBDA / (l + 1) + 1.0))
        part1 = sc_spmv(xs1a, xs1b, s1, d1)
        part2 = sc_spmv(xs2a, xs2b, s2, d2)
        xs1a, xs1b = _tc_update(part1, dir1, dor1, x, Ws1[l],
                                gs1[l], bs1[l], beta, NP, RB)
        xs2a, xs2b = _tc_update(part2, dir2, dor2, x, Ws2[l],
                                gs2[l], bs2[l], beta, NP, RB)

    beta = float(np.log(LAMBDA / n_layers + 1.0))
    part1 = sc_spmv(xs1a, xs1b, s1, d1)
    part2 = sc_spmv(xs2a, xs2b, s2, d2)
    h1, h2, ws = _tc_scores(part1, part2, dir1, dir2, x,
                            Ws1[2], Ws2[2], att_W,
                            att_b.reshape(1, -1), att_q.reshape(-1, 1),
                            beta, N, NP, RB)
    return _tc_combine(h1, h2, ws, N, RB)


# speedup vs baseline: 1.4671x; 1.0015x over previous
import functools

import jax
import jax.numpy as jnp
import numpy as np
from jax import lax
from jax.experimental import pallas as pl
from jax.experimental.pallas import tpu as pltpu
from jax.experimental.pallas import tpu_sc as plsc

ALPHA = 0.2
LAMBDA = 1.0
BN_INV = float(1.0 / np.sqrt(1.0 + 1e-5))

NSUB = 16
LANE = 128
KB = 8

_SC_PARAMS = pltpu.CompilerParams(use_tc_tiling_on_sc=False)


def _fill_1d(ref, n, val):
    def body(k, _):
        ref[pl.ds(k * 16, 16)] = jnp.full((16,), val, jnp.float32)
        return _
    lax.fori_loop(0, n // 16, body, None)


def _fill_2d(ref, rows, cols, val):
    def body(r, _):
        for j in range(cols // 16):
            ref[r, pl.ds(j * 16, 16)] = jnp.full((16,), val, jnp.float32)
        return _
    lax.fori_loop(0, rows, body, None)


def _make_sc_degrees(NP, ROWS_T):
    STRIPE = NP // NSUB
    n_outer = ROWS_T // KB
    mesh = plsc.VectorSubcoreMesh(core_axis_name="c", subcore_axis_name="s")
    cnt_t = jax.ShapeDtypeStruct((NP,), jnp.float32)

    @functools.partial(
        pl.kernel,
        out_type=[cnt_t, cnt_t, cnt_t, cnt_t],
        mesh=mesh,
        compiler_params=_SC_PARAMS,
        scratch_types=[
            pltpu.VMEM((KB, LANE), jnp.int32),
            pltpu.VMEM((KB, LANE), jnp.int32),
            pltpu.VMEM((LANE,), jnp.float32),
            pltpu.VMEM((STRIPE,), jnp.float32),
            pltpu.VMEM_SHARED((NP,), jnp.float32),
            pltpu.VMEM_SHARED((NP,), jnp.float32),
            pltpu.SemaphoreType.DMA,
        ],
    )
    def k(s1_h, d1_h, s2_h, d2_h, co1_h, ci1_h, co2_h, ci2_h,
          sidx_v, didx_v, ones_v, zb_v, acc_o, acc_i, dsem):
        c = lax.axis_index("c")
        t = lax.axis_index("s")
        _fill_1d(ones_v, LANE, 1.0)
        _fill_1d(zb_v, STRIPE, 0.0)
        pltpu.sync_copy(zb_v, acc_o.at[pl.ds(STRIPE * t, STRIPE)])
        pltpu.sync_copy(zb_v, acc_i.at[pl.ds(STRIPE * t, STRIPE)])
        plsc.subcore_barrier()

        def run(s_h, d_h, co_h, ci_h):
            def outer(i, carry):
                rb = t * ROWS_T + i * KB
                pltpu.sync_copy(s_h.at[pl.ds(rb, KB)], sidx_v)
                pltpu.sync_copy(d_h.at[pl.ds(rb, KB)], didx_v)
                for j in range(KB):
                    pltpu.async_copy(ones_v, acc_o.at[sidx_v.at[j]], dsem,
                                     add=True)
                for j in range(KB):
                    pltpu.async_copy(ones_v, acc_i.at[didx_v.at[j]], dsem,
                                     add=True)
                for j in range(KB):
                    pltpu.make_async_copy(
                        ones_v, acc_o.at[sidx_v.at[j]], dsem).wait()
                    pltpu.make_async_copy(
                        ones_v, acc_i.at[didx_v.at[j]], dsem).wait()
                return carry
            lax.fori_loop(0, n_outer, outer, 0)
            plsc.subcore_barrier()
            pltpu.sync_copy(acc_o.at[pl.ds(STRIPE * t, STRIPE)],
                            co_h.at[pl.ds(STRIPE * t, STRIPE)])
            pltpu.sync_copy(acc_i.at[pl.ds(STRIPE * t, STRIPE)],
                            ci_h.at[pl.ds(STRIPE * t, STRIPE)])

        @pl.when(c == 0)
        def _():
            run(s1_h, d1_h, co1_h, ci1_h)

        @pl.when(c == 1)
        def _():
            run(s2_h, d2_h, co2_h, ci2_h)

    return k


def _make_sc_spmv(NP, D, ROWS_T):
    DH = D // 2
    STRIPE = NP // NSUB
    PH = 4
    PROWS = ROWS_T // PH
    NGRP = PROWS // 4
    mesh = plsc.VectorSubcoreMesh(core_axis_name="c", subcore_axis_name="s")

    @functools.partial(
        pl.kernel,
        out_type=jax.ShapeDtypeStruct((2, NP, DH), jnp.float32),
        mesh=mesh,
        compiler_params=_SC_PARAMS,
        scratch_types=(
            [pltpu.VMEM((PROWS, LANE), jnp.int32),
             pltpu.VMEM((PROWS, LANE), jnp.int32),
             pltpu.VMEM((4 * LANE, DH), jnp.float32),
             pltpu.VMEM_SHARED((NP, DH), jnp.float32),
             pltpu.VMEM_SHARED((NP, DH), jnp.float32)]
            + [pltpu.SemaphoreType.DMA] * 8
        ),
    )
    def k(xsa_h, xsb_h, s_h, d_h, out_h,
          sidx_v, didx_v, rows_v, sxs, sacc,
          g0, g1, g2, g3, ss0, ss1, ss2, ss3):
        c = lax.axis_index("c")
        t = lax.axis_index("s")
        gsem = [g0, g1, g2, g3]
        ssem = [ss0, ss1, ss2, ss3]

        def slot(k_):
            return rows_v.at[pl.ds(k_ * LANE, LANE)]

        def gi(r, k_):
            pltpu.async_copy(sxs.at[sidx_v.at[r]], slot(k_), gsem[k_])

        def gw(r, k_):
            pltpu.make_async_copy(sxs.at[sidx_v.at[r]], slot(k_),
                                  gsem[k_]).wait()

        def si(r, k_):
            pltpu.async_copy(slot(k_), sacc.at[didx_v.at[r]], ssem[k_],
                             add=True)

        def sw(r, k_):
            pltpu.make_async_copy(slot(k_), sacc.at[didx_v.at[r]],
                                  ssem[k_]).wait()

        def run(xs_h, half):
            pltpu.sync_copy(xs_h.at[pl.ds(STRIPE * t, STRIPE)],
                            sxs.at[pl.ds(STRIPE * t, STRIPE)])
            _fill_2d(rows_v, LANE, DH, 0.0)
            for q in range(STRIPE // LANE):
                pltpu.sync_copy(
                    rows_v.at[pl.ds(0, LANE)],
                    sacc.at[pl.ds(STRIPE * t + LANE * q, LANE)])
            plsc.subcore_barrier()

            def phase(ph, carry):
                rb = t * ROWS_T + ph * PROWS
                pltpu.sync_copy(s_h.at[pl.ds(rb, PROWS)], sidx_v)
                pltpu.sync_copy(d_h.at[pl.ds(rb, PROWS)], didx_v)
                gi(0, 0)
                gi(1, 1)
                gi(2, 2)
                gw(0, 0)
                si(0, 0)
                gi(3, 3)
                gw(1, 1)
                si(1, 1)

                def group(g, cr):
                    for k_ in range(4):
                        r = g * 4 + k_
                        sw(r - 4, k_)
                        gi(r, k_)
                        k2 = (k_ + 2) % 4
                        gw(r - 2, k2)
                        si(r - 2, k2)
                    return cr
                lax.fori_loop(1, NGRP, group, 0)
                gw(PROWS - 2, 2)
                si(PROWS - 2, 2)
                gw(PROWS - 1, 3)
                si(PROWS - 1, 3)
                for k_ in range(4):
                    sw(0, k_)
                return carry
            lax.fori_loop(0, PH, phase, 0)
            plsc.subcore_barrier()
            pltpu.sync_copy(sacc.at[pl.ds(STRIPE * t, STRIPE)],
                            out_h.at[half, pl.ds(STRIPE * t, STRIPE)])

        @pl.when(c == 0)
        def _():
            run(xsa_h, 0)

        @pl.when(c == 1)
        def _():
            run(xsb_h, 1)

    return k


def _row_block_spec(RB, D):
    return pl.BlockSpec((RB, D), lambda i: (i, 0))


def _full_spec(shape):
    return pl.BlockSpec(shape, lambda i: tuple(0 for _ in shape))


def _tc_prep(xp, co1, ci1, co2, ci2, NP, RB):
    D = xp.shape[1]
    grid = (NP // RB,)

    DH = D // 2

    def body(x_ref, co1_ref, ci1_ref, co2_ref, ci2_ref,
             xs1a_ref, xs1b_ref, xs2a_ref, xs2b_ref,
             dir1_ref, dir2_ref, dor1_ref, dor2_ref):
        x = x_ref[...]
        dor1 = lax.rsqrt(jnp.maximum(co1_ref[...], 1.0))
        dor2 = lax.rsqrt(jnp.maximum(co2_ref[...], 1.0))
        dir1_ref[...] = lax.rsqrt(jnp.maximum(ci1_ref[...], 1.0))
        dir2_ref[...] = lax.rsqrt(jnp.maximum(ci2_ref[...], 1.0))
        dor1_ref[...] = dor1
        dor2_ref[...] = dor2
        xs1 = x * dor1
        xs2 = x * dor2
        xs1a_ref[...] = xs1[:, :DH]
        xs1b_ref[...] = xs1[:, DH:]
        xs2a_ref[...] = xs2[:, :DH]
        xs2b_ref[...] = xs2[:, DH:]

    col = pl.BlockSpec((RB, 1), lambda i: (i, 0))
    halfspec = _row_block_spec(RB, DH)
    halfshape = jax.ShapeDtypeStruct((NP, DH), jnp.float32)
    return pl.pallas_call(
        body,
        grid=grid,
        in_specs=[_row_block_spec(RB, D), col, col, col, col],
        out_specs=[halfspec, halfspec, halfspec, halfspec,
                   col, col, col, col],
        out_shape=[
            halfshape, halfshape, halfshape, halfshape,
            jax.ShapeDtypeStruct((NP, 1), jnp.float32),
            jax.ShapeDtypeStruct((NP, 1), jnp.float32),
            jax.ShapeDtypeStruct((NP, 1), jnp.float32),
            jax.ShapeDtypeStruct((NP, 1), jnp.float32),
        ],
    )(xp, co1, ci1, co2, ci2)


def _tc_update(part, dirr, dorr, xp, W, g, b, beta, NP, RB):
    D = xp.shape[1]
    DH = D // 2
    grid = (NP // RB,)

    def body(pa_ref, pb_ref, dir_ref, dor_ref, x_ref,
             W_ref, g_ref, b_ref, oa_ref, ob_ref):
        x = x_ref[...]
        p = jnp.concatenate([pa_ref[...].reshape(RB, DH),
                             pb_ref[...].reshape(RB, DH)], axis=1)
        h = (1.0 - ALPHA) * p * dir_ref[...] + ALPHA * x
        hw = jnp.dot(h, W_ref[...], preferred_element_type=jnp.float32)
        tt = (1.0 - beta) * h + beta * hw
        tt = tt * BN_INV * g_ref[...] + b_ref[...]
        tt = jnp.maximum(tt, 0.0)
        tt = tt * dor_ref[...]
        oa_ref[...] = tt[:, :DH]
        ob_ref[...] = tt[:, DH:]

    col = pl.BlockSpec((RB, 1), lambda i: (i, 0))
    row = _row_block_spec(RB, D)
    half = _row_block_spec(RB, DH)
    halfshape = jax.ShapeDtypeStruct((NP, DH), jnp.float32)

    def pspec(hi):
        return pl.BlockSpec((1, RB, DH), lambda i: (hi, i, 0))

    return pl.pallas_call(
        body,
        grid=grid,
        in_specs=[pspec(0), pspec(1), col, col, row,
                  _full_spec((D, D)), _full_spec((1, D)), _full_spec((1, D))],
        out_specs=[half, half],
        out_shape=[halfshape, halfshape],
    )(part, part, dirr, dorr, xp, W, g, b)


def _tc_scores(part1, part2, dir1, dir2, xp,
               W1, W2, attW, attb, attq, beta, N, NP, RB):
    D = xp.shape[1]
    DH = D // 2
    grid = (NP // RB,)

    def body(p1a_ref, p1b_ref, p2a_ref, p2b_ref,
             dir1_ref, dir2_ref, x_ref, W1_ref, W2_ref,
             attW_ref, attb_ref, attq_ref, h1_ref, h2_ref, ws_ref):
        i = pl.program_id(0)
        x = x_ref[...]

        def one(pa_ref, pb_ref, dirr, W_ref, h_ref):
            p = jnp.concatenate([pa_ref[...].reshape(RB, DH),
                                 pb_ref[...].reshape(RB, DH)], axis=1)
            h = (1.0 - ALPHA) * p * dirr[...] + ALPHA * x
            hw = jnp.dot(h, W_ref[...], preferred_element_type=jnp.float32)
            h = (1.0 - beta) * h + beta * hw
            h_ref[...] = h
            tt = jnp.tanh(jnp.dot(h, attW_ref[...],
                                  preferred_element_type=jnp.float32)
                          + attb_ref[...])
            return jnp.dot(tt, attq_ref[...],
                           preferred_element_type=jnp.float32)

        w1 = one(p1a_ref, p1b_ref, dir1_ref, W1_ref, h1_ref)
        w2 = one(p2a_ref, p2b_ref, dir2_ref, W2_ref, h2_ref)
        rowid = i * RB + lax.broadcasted_iota(jnp.int32, (RB, 1), 0)
        valid = rowid < N
        s1 = jnp.sum(jnp.where(valid, w1, 0.0))
        s2 = jnp.sum(jnp.where(valid, w2, 0.0))
        rr = lax.broadcasted_iota(jnp.int32, (8, 128), 0)
        cc = lax.broadcasted_iota(jnp.int32, (8, 128), 1)
        contrib = (jnp.where((rr == 0) & (cc == 0), s1, 0.0)
                   + jnp.where((rr == 0) & (cc == 1), s2, 0.0))

        @pl.when(i == 0)
        def _():
            ws_ref[...] = contrib

        @pl.when(i > 0)
        def _():
            ws_ref[...] = ws_ref[...] + contrib

    col = pl.BlockSpec((RB, 1), lambda i: (i, 0))
    row = _row_block_spec(RB, D)

    def pspec(hi):
        return pl.BlockSpec((1, RB, DH), lambda i: (hi, i, 0))

    return pl.pallas_call(
        body,
        grid=grid,
        in_specs=[pspec(0), pspec(1), pspec(0), pspec(1),
                  col, col, row,
                  _full_spec((D, D)), _full_spec((D, D)),
                  _full_spec(attW.shape), _full_spec((1, attW.shape[1])),
                  _full_spec((attW.shape[1], 1))],
        out_specs=[row, row, pl.BlockSpec((8, 128), lambda i: (0, 0))],
        out_shape=[
            jax.ShapeDtypeStruct((NP, D), jnp.float32),
            jax.ShapeDtypeStruct((NP, D), jnp.float32),
            jax.ShapeDtypeStruct((8, 128), jnp.float32),
        ],
    )(part1, part1, part2, part2, dir1, dir2, xp, W1, W2, attW, attb, attq)


def _tc_combine(h1, h2, ws, N, RB):
    NP, D = h1.shape
    grid = (pl.cdiv(N, RB),)

    def body(h1_ref, h2_ref, ws_ref, o_ref):
        ws = ws_ref[...]
        rr = lax.broadcasted_iota(jnp.int32, (8, 128), 0)
        cc = lax.broadcasted_iota(jnp.int32, (8, 128), 1)
        s1 = jnp.sum(jnp.where((rr == 0) & (cc == 0), ws, 0.0))
        s2 = jnp.sum(jnp.where((rr == 0) & (cc == 1), ws, 0.0))
        m1 = s1 / N
        m2 = s2 / N
        mx = jnp.maximum(m1, m2)
        e1 = jnp.exp(m1 - mx)
        e2 = jnp.exp(m2 - mx)
        bb1 = e1 / (e1 + e2)
        bb2 = e2 / (e1 + e2)
        o_ref[...] = bb1 * h1_ref[...] + bb2 * h2_ref[...]

    row = _row_block_spec(RB, D)
    return pl.pallas_call(
        body,
        grid=grid,
        in_specs=[row, row, pl.BlockSpec((8, 128), lambda i: (0, 0))],
        out_specs=row,
        out_shape=jax.ShapeDtypeStruct((N, D), jnp.float32),
    )(h1, h2, ws)


def kernel(x, edge_index_mp1, edge_index_mp2,
           W_mp1_0, W_mp1_1, W_mp1_2, bn_g_mp1_0, bn_b_mp1_0, bn_g_mp1_1, bn_b_mp1_1,
           W_mp2_0, W_mp2_1, W_mp2_2, bn_g_mp2_0, bn_b_mp2_0, bn_g_mp2_1, bn_b_mp2_1,
           att_W, att_b, att_q):
    N, D = x.shape
    E = edge_index_mp1.shape[1]
    n_layers = 3
    RB = 1024

    NP = ((N + 1 + NSUB * LANE - 1) // (NSUB * LANE)) * (NSUB * LANE)
    CH = KB * LANE
    EPT = ((E + NSUB - 1) // NSUB + CH - 1) // CH * CH
    EP = EPT * NSUB
    ROWS_T = EPT // LANE

    def prep_edges(ei):
        pad = jnp.full((EP - E,), N, jnp.int32)
        s = jnp.concatenate([ei[0], pad]).reshape(EP // LANE, LANE)
        d = jnp.concatenate([ei[1], pad]).reshape(EP // LANE, LANE)
        return s, d

    s1, d1 = prep_edges(edge_index_mp1)
    s2, d2 = prep_edges(edge_index_mp2)

    sc_deg = _make_sc_degrees(NP, ROWS_T)
    sc_spmv = _make_sc_spmv(NP, D, ROWS_T)

    co1, ci1, co2, ci2 = sc_deg(s1, d1, s2, d2)
    co1, ci1, co2, ci2 = (v[:, None] for v in (co1, ci1, co2, ci2))

    (xs1a, xs1b, xs2a, xs2b,
     dir1, dir2, dor1, dor2) = _tc_prep(x, co1, ci1, co2, ci2, NP, RB)

    Ws1 = [W_mp1_0, W_mp1_1, W_mp1_2]
    Ws2 = [W_mp2_0, W_mp2_1, W_mp2_2]
    gs1 = [bn_g_mp1_0.reshape(1, D), bn_g_mp1_1.reshape(1, D)]
    bs1 = [bn_b_mp1_0.reshape(1, D), bn_b_mp1_1.reshape(1, D)]
    gs2 = [bn_g_mp2_0.reshape(1, D), bn_g_mp2_1.reshape(1, D)]
    bs2 = [bn_b_mp2_0.reshape(1, D), bn_b_mp2_1.reshape(1, D)]

    for l in range(n_layers - 1):
        beta = float(np.log(LAMBDA / (l + 1) + 1.0))
        part1 = sc_spmv(xs1a, xs1b, s1, d1)
        part2 = sc_spmv(xs2a, xs2b, s2, d2)
        xs1a, xs1b = _tc_update(part1, dir1, dor1, x, Ws1[l],
                                gs1[l], bs1[l], beta, NP, RB)
        xs2a, xs2b = _tc_update(part2, dir2, dor2, x, Ws2[l],
                                gs2[l], bs2[l], beta, NP, RB)

    beta = float(np.log(LAMBDA / n_layers + 1.0))
    part1 = sc_spmv(xs1a, xs1b, s1, d1)
    part2 = sc_spmv(xs2a, xs2b, s2, d2)
    h1, h2, ws = _tc_scores(part1, part2, dir1, dir2, x,
                            Ws1[2], Ws2[2], att_W,
                            att_b.reshape(1, -1), att_q.reshape(-1, 1),
                            beta, N, NP, RB)
    return _tc_combine(h1, h2, ws, N, RB)
